# baseline (speedup 1.0000x reference)
import jax
import jax.numpy as jnp
from jax import lax
from jax.experimental import pallas as pl
from jax.experimental.pallas import tpu as pltpu

_BB = 1024


def _body(w_ref, adj_ref, bq_ref, bs_ref, fp_ref, qos_ref, fh_ref,
          trust_ref, cons_ref, supp_ref, stab_ref):
    f32 = jnp.float32
    N = adj_ref.shape[0]
    W = fh_ref.shape[0] // N
    S = 8

    fp = fp_ref[...]
    m = (adj_ref[...] > 0).astype(f32)
    counts = jnp.sum(m, axis=0, keepdims=True)
    pf = (fp > 0.5).astype(f32)
    numer = lax.dot_general(m, pf, (((0,), (0,)), ((), ())),
                            preferred_element_type=f32)
    mpf = numer / jnp.maximum(counts.T, 1.0)
    consistent = (mpf <= fp + 0.3).astype(f32)
    cons = jnp.where(counts.T > 0, consistent, 1.0)

    w1 = w_ref[0]
    w2 = w_ref[1]
    w3 = w_ref[2]
    inv_w = f32(1.0 / W)

    for nb in range(N // S):
        lo = nb * S
        v = fh_ref[pl.ds(lo, S), :]
        a1 = v
        a2 = v * v
        for wi in range(1, W):
            v = fh_ref[pl.ds(wi * N + lo, S), :]
            a1 = a1 + v
            a2 = a2 + v * v
        mean = a1 * inv_w
        var = a2 * inv_w - mean * mean
        stab = 1.0 / (1.0 + var)

        qn = (qos_ref[pl.ds(lo, S)] - bq_ref[...]) * bs_ref[...]
        nsq = jnp.sum(qn * qn, axis=1)
        supp = jax.nn.sigmoid(jnp.sqrt(nsq))

        cs = cons[lo:lo + S, :]
        trust = w1 * cs + w2 * supp + w3 * stab
        nt = supp.shape[1] // 128
        trust_ref[pl.ds(lo, S)] = trust.reshape(S, nt, 128)
        cons_ref[pl.ds(lo, S)] = cs.reshape(S, nt, 128)
        supp_ref[pl.ds(lo, S)] = supp.reshape(S, nt, 128)
        stab_ref[pl.ds(lo, S)] = stab.reshape(S, nt, 128)


def kernel(fault_probs, qos_observations, fault_history, adjacency_matrix,
           gamma1, gamma2, gamma3, baseline_qos, baseline_std):
    B, N, W = fault_history.shape
    Q = qos_observations.shape[-1]
    fh_lin = fault_history.transpose(2, 1, 0).reshape(W * N, B)
    qos3 = qos_observations.transpose(1, 2, 0)
    fp2 = fault_probs.transpose(1, 2, 0).reshape(N, B)

    gsum = gamma1 + gamma2 + gamma3 + 1e-8
    w = jnp.stack([gamma1 / gsum, gamma2 / gsum, gamma3 / gsum]).astype(jnp.float32)
    bq = baseline_qos[None, :, None]
    bs = (1.0 / (baseline_std + 1e-8))[None, :, None]

    grid = (B // _BB,)
    out_shape = jax.ShapeDtypeStruct((N, B // 128, 128), jnp.float32)

    trust, cons, supp, stab = pl.pallas_call(
        _body,
        grid=grid,
        in_specs=[
            pl.BlockSpec(memory_space=pltpu.SMEM),
            pl.BlockSpec((N, N), lambda j: (0, 0)),
            pl.BlockSpec((1, Q, 1), lambda j: (0, 0, 0)),
            pl.BlockSpec((1, Q, 1), lambda j: (0, 0, 0)),
            pl.BlockSpec((N, _BB), lambda j: (0, j)),
            pl.BlockSpec((N, Q, _BB), lambda j: (0, 0, j)),
            pl.BlockSpec((W * N, _BB), lambda j: (0, j)),
        ],
        out_specs=[pl.BlockSpec((N, _BB // 128, 128), lambda j: (0, j, 0))] * 4,
        out_shape=[out_shape] * 4,
        compiler_params=pltpu.CompilerParams(
            dimension_semantics=("arbitrary",),
        ),
    )(w, adjacency_matrix, bq, bs, fp2, qos3, fh_lin)

    def back(a):
        return a.transpose(1, 2, 0).reshape(B, N)[:, :, None]

    return back(trust), back(cons), back(supp), back(stab)


# speedup vs baseline: 1.0127x; 1.0127x over previous
import jax
import jax.numpy as jnp
from jax import lax
from jax.experimental import pallas as pl
from jax.experimental.pallas import tpu as pltpu

_BB = 2048


def _body(w_ref, adj_ref, bq_ref, bs_ref, fp_ref, qos_ref, fh_ref,
          trust_ref, cons_ref, supp_ref, stab_ref):
    f32 = jnp.float32
    N = adj_ref.shape[0]
    W = fh_ref.shape[0] // N
    S = 8

    fp = fp_ref[...]
    m = (adj_ref[...] > 0).astype(f32)
    counts = jnp.sum(m, axis=0, keepdims=True)
    pf = (fp > 0.5).astype(f32)
    numer = lax.dot_general(m, pf, (((0,), (0,)), ((), ())),
                            preferred_element_type=f32)
    mpf = numer / jnp.maximum(counts.T, 1.0)
    consistent = (mpf <= fp + 0.3).astype(f32)
    cons = jnp.where(counts.T > 0, consistent, 1.0)

    w1 = w_ref[0]
    w2 = w_ref[1]
    w3 = w_ref[2]
    inv_w = f32(1.0 / W)

    for nb in range(N // S):
        lo = nb * S
        v = fh_ref[pl.ds(lo, S), :]
        a1 = v
        a2 = v * v
        for wi in range(1, W):
            v = fh_ref[pl.ds(wi * N + lo, S), :]
            a1 = a1 + v
            a2 = a2 + v * v
        mean = a1 * inv_w
        var = a2 * inv_w - mean * mean
        stab = 1.0 / (1.0 + var)

        qn = (qos_ref[pl.ds(lo, S)] - bq_ref[...]) * bs_ref[...]
        nsq = jnp.sum(qn * qn, axis=1)
        supp = jax.nn.sigmoid(jnp.sqrt(nsq))

        cs = cons[lo:lo + S, :]
        trust = w1 * cs + w2 * supp + w3 * stab
        nt = supp.shape[1] // 128
        trust_ref[pl.ds(lo, S)] = trust.reshape(S, nt, 128)
        cons_ref[pl.ds(lo, S)] = cs.reshape(S, nt, 128)
        supp_ref[pl.ds(lo, S)] = supp.reshape(S, nt, 128)
        stab_ref[pl.ds(lo, S)] = stab.reshape(S, nt, 128)


def kernel(fault_probs, qos_observations, fault_history, adjacency_matrix,
           gamma1, gamma2, gamma3, baseline_qos, baseline_std):
    B, N, W = fault_history.shape
    Q = qos_observations.shape[-1]
    fh_lin = fault_history.transpose(2, 1, 0).reshape(W * N, B)
    qos3 = qos_observations.transpose(1, 2, 0)
    fp2 = fault_probs.transpose(1, 2, 0).reshape(N, B)

    gsum = gamma1 + gamma2 + gamma3 + 1e-8
    w = jnp.stack([gamma1 / gsum, gamma2 / gsum, gamma3 / gsum]).astype(jnp.float32)
    bq = baseline_qos[None, :, None]
    bs = (1.0 / (baseline_std + 1e-8))[None, :, None]

    grid = (B // _BB,)
    out_shape = jax.ShapeDtypeStruct((N, B // 128, 128), jnp.float32)

    trust, cons, supp, stab = pl.pallas_call(
        _body,
        grid=grid,
        in_specs=[
            pl.BlockSpec(memory_space=pltpu.SMEM),
            pl.BlockSpec((N, N), lambda j: (0, 0)),
            pl.BlockSpec((1, Q, 1), lambda j: (0, 0, 0)),
            pl.BlockSpec((1, Q, 1), lambda j: (0, 0, 0)),
            pl.BlockSpec((N, _BB), lambda j: (0, j)),
            pl.BlockSpec((N, Q, _BB), lambda j: (0, 0, j)),
            pl.BlockSpec((W * N, _BB), lambda j: (0, j)),
        ],
        out_specs=[pl.BlockSpec((N, _BB // 128, 128), lambda j: (0, j, 0))] * 4,
        out_shape=[out_shape] * 4,
        compiler_params=pltpu.CompilerParams(
            dimension_semantics=("arbitrary",),
        ),
    )(w, adjacency_matrix, bq, bs, fp2, qos3, fh_lin)

    def back(a):
        return a.transpose(1, 2, 0).reshape(B, N)[:, :, None]

    return back(trust), back(cons), back(supp), back(stab)
